# baseline (speedup 1.0000x reference)
import jax
import jax.numpy as jnp
from jax.experimental import pallas as pl
from jax.experimental.pallas import tpu as pltpu


def _pos_kernel(col_ref, row_ref, out_ref, scr, sem):
    h, w = scr.shape[0], scr.shape[1]
    d = scr.shape[2] // 2
    b = out_ref.shape[0]
    col = col_ref[0:w, :]
    row = row_ref[0:h, :]
    hh = h // 2
    scr[0:hh, :, 0:d] = jnp.broadcast_to(col[None, :, :], (hh, w, d))
    scr[0:hh, :, d:2 * d] = jnp.broadcast_to(row[0:hh, None, :], (hh, w, d))
    first = [
        pltpu.make_async_copy(
            scr.at[pl.ds(0, hh)], out_ref.at[i, pl.ds(0, hh)], sem.at[0, i]
        )
        for i in range(b)
    ]
    for cp in first:
        cp.start()
    scr[hh:h, :, 0:d] = jnp.broadcast_to(col[None, :, :], (h - hh, w, d))
    scr[hh:h, :, d:2 * d] = jnp.broadcast_to(
        row[hh:h, None, :], (h - hh, w, d)
    )
    second = [
        pltpu.make_async_copy(
            scr.at[pl.ds(hh, h - hh)], out_ref.at[i, pl.ds(hh, h - hh)],
            sem.at[1, i]
        )
        for i in range(b)
    ]
    for cp in second:
        cp.start()
    for cp in first + second:
        cp.wait()


def kernel(tensor_list, row_embed, col_embed):
    b = tensor_list.shape[0]
    h, w = tensor_list.shape[-2], tensor_list.shape[-1]
    d = col_embed.shape[-1]
    out = pl.pallas_call(
        _pos_kernel,
        out_shape=jax.ShapeDtypeStruct((b, h, w, 2 * d), jnp.float32),
        in_specs=[
            pl.BlockSpec(memory_space=pltpu.VMEM),
            pl.BlockSpec(memory_space=pltpu.VMEM),
        ],
        out_specs=pl.BlockSpec(memory_space=pl.ANY),
        scratch_shapes=[
            pltpu.VMEM((h, w, 2 * d), jnp.float32),
            pltpu.SemaphoreType.DMA((2, b)),
        ],
    )(col_embed, row_embed)
    return jnp.transpose(out, (0, 3, 1, 2))


# speedup vs baseline: 1.0015x; 1.0015x over previous
import jax
import jax.numpy as jnp
from jax.experimental import pallas as pl
from jax.experimental.pallas import tpu as pltpu


def _pos_kernel(col_ref, row_ref, out_ref, scr, sem):
    h, w = scr.shape[0], scr.shape[1]
    d = scr.shape[2] // 2
    b = out_ref.shape[0]
    col = col_ref[0:w, :]
    row = row_ref[0:h, :]
    scr[:, :, 0:d] = jnp.broadcast_to(col[None, :, :], (h, w, d))
    scr[:, :, d:2 * d] = jnp.broadcast_to(row[:, None, :], (h, w, d))
    copies = [
        pltpu.make_async_copy(scr, out_ref.at[i], sem.at[i]) for i in range(b)
    ]
    for cp in copies:
        cp.start()
    for cp in copies:
        cp.wait()


def kernel(tensor_list, row_embed, col_embed):
    b = tensor_list.shape[0]
    h, w = tensor_list.shape[-2], tensor_list.shape[-1]
    d = col_embed.shape[-1]
    out = pl.pallas_call(
        _pos_kernel,
        out_shape=jax.ShapeDtypeStruct((b, h, w, 2 * d), jnp.float32),
        in_specs=[
            pl.BlockSpec(memory_space=pltpu.VMEM),
            pl.BlockSpec(memory_space=pltpu.VMEM),
        ],
        out_specs=pl.BlockSpec(memory_space=pl.ANY),
        scratch_shapes=[
            pltpu.VMEM((h, w, 2 * d), jnp.float32),
            pltpu.SemaphoreType.DMA((b,)),
        ],
    )(col_embed, row_embed)
    return jnp.transpose(out, (0, 3, 1, 2))
